# Initial kernel scaffold; baseline (speedup 1.0000x reference)
import jax
import jax.numpy as jnp
from jax.experimental import pallas as pl


def kernel(x_ind, x_org, x_ext, ei_ind_txn_ind, ei_org_txn_ind, ei_ext_txn_ind, ei_ind_txn_org, ei_org_txn_org, ei_ext_txn_org, ei_ind_role_org, ei_ind_rev_txn_ind, ei_org_rev_txn_ind, ei_ext_rev_txn_ind, ei_ind_rev_txn_org, ei_org_rev_txn_org, ei_ext_rev_txn_org, ei_org_rev_role_ind, edge_attr_dummy, Wl, bl, Wr):
    raise NotImplementedError("write your pallas kernel here")


# speedup vs baseline: 9.7945x; 9.7945x over previous
import functools

import jax
import jax.numpy as jnp
from jax import lax
from jax.experimental import pallas as pl
from jax.experimental.pallas import tpu as pltpu
from jax.experimental.pallas import tpu_sc as plsc

_SRC = ["ind", "org", "ext", "ind", "org", "ext", "ind",
        "ind", "org", "ext", "ind", "org", "ext", "org"]
_DST = ["ind", "ind", "ind", "org", "org", "org", "org",
        "ind", "ind", "ind", "org", "org", "org", "ind"]
_NREL = 14

_SRC_COL = {}
for _t in ("ind", "org", "ext"):
    for _c, _r in enumerate([i for i in range(_NREL) if _SRC[i] == _t]):
        _SRC_COL[_r] = _c
_DST_SLOT = {}
for _t in ("ind", "org"):
    for _c, _r in enumerate([i for i in range(_NREL) if _DST[i] == _t]):
        _DST_SLOT[_r] = _c
_CORE = {r: (0 if _DST[r] == "ind" else 1) for r in range(_NREL)}

_E = 40000
_CH = 128
_NFULL = _E // _CH
_TAIL = _E - _NFULL * _CH
_NSUB = 16
_KMAX = -(-_NFULL // _NSUB)
_NP = 106496
_ZCH = 8192


def _project_kernel(a_ref, x_ref, o_ref):
    o_ref[...] = lax.dot_general(
        a_ref[...], x_ref[...], (((1,), (1,)), ((), ())),
        preferred_element_type=jnp.float32)


def _project(x, at, bn=2048):
    n = x.shape[0]
    grid = -(-n // bn)
    return pl.pallas_call(
        _project_kernel,
        grid=(grid,),
        in_specs=[
            pl.BlockSpec((8, 128), lambda i: (0, 0)),
            pl.BlockSpec((bn, 128), lambda i: (i, 0)),
        ],
        out_specs=pl.BlockSpec((8, bn), lambda i: (0, i)),
        out_shape=jax.ShapeDtypeStruct((8, n), jnp.float32),
    )(at, x)


def _combine_kernel(*refs):
    sums = refs[0:7]
    cnts = refs[7:14]
    y_ref, b_ref, o_ref = refs[14], refs[15], refs[16]
    tot = y_ref[...] + b_ref[0, 0]
    for j in range(7):
        tot = tot + sums[j][...] / jnp.maximum(cnts[j][...], 1.0)
    o_ref[...] = jax.nn.sigmoid(tot)


def _combine(sums, cnts, z, bsum, bn=1024):
    n = z.shape[0]
    grid = -(-n // bn)
    vec = pl.BlockSpec((bn,), lambda i: (i,))
    return pl.pallas_call(
        _combine_kernel,
        grid=(grid,),
        in_specs=[vec] * 15 + [pl.BlockSpec(memory_space=pltpu.SMEM)],
        out_specs=vec,
        out_shape=jax.ShapeDtypeStruct((n,), jnp.float32),
    )(*sums, *cnts, z, bsum)


def _segment_body(*refs):
    ys = refs[0:_NREL]
    eis = refs[_NREL:2 * _NREL]
    zeros_hbm = refs[2 * _NREL]
    ones_hbm = refs[2 * _NREL + 1]
    o_sum_ind = refs[30:37]
    o_cnt_ind = refs[37:44]
    o_sum_org = refs[44:51]
    o_cnt_org = refs[51:58]
    sc = refs[58:]
    sums = sc[0:7]
    cnts = sc[7:14]
    idx0, idx1, vals, ones, idx0t, idx1t, valst, onest = sc[14:22]
    zbuf, wbuf = sc[22:24]
    gsem, ssem, csem = sc[24:27]

    c = lax.axis_index("c")
    s = lax.axis_index("s")

    pltpu.sync_copy(ones_hbm, ones)
    pltpu.sync_copy(ones_hbm.at[pl.ds(0, _TAIL)], onest)
    pltpu.sync_copy(zeros_hbm, zbuf)
    for a, acc in enumerate(sums + cnts):
        @pl.loop(0, _NP // _ZCH)
        def _(i):
            @pl.when(((a * (_NP // _ZCH) + i) % _NSUB) == s)
            def _():
                pltpu.sync_copy(zbuf, acc.at[pl.ds(i * _ZCH, _ZCH)])
    plsc.subcore_barrier()

    for r in range(_NREL):
        y = ys[r]
        ei = eis[r]
        sum_r = sums[_DST_SLOT[r]]
        cnt_r = cnts[_DST_SLOT[r]]
        on_core = c == _CORE[r]

        @pl.when(on_core)
        def _():
            @pl.loop(0, _KMAX)
            def _(k):
                j = k * _NSUB + s

                @pl.when(j < _NFULL)
                def _():
                    off = j * _CH
                    pltpu.sync_copy(ei.at[0, pl.ds(off, _CH)], idx0)
                    pltpu.sync_copy(ei.at[1, pl.ds(off, _CH)], idx1)
                    pltpu.async_copy(y.at[idx0], vals, gsem).wait()
                    pltpu.async_copy(vals, sum_r.at[idx1], ssem,
                                     add=True).wait()
                    pltpu.async_copy(ones, cnt_r.at[idx1], csem,
                                     add=True).wait()

        @pl.when(on_core & (s == r))
        def _():
            off = _NFULL * _CH
            pltpu.sync_copy(ei.at[0, pl.ds(off, _TAIL)], idx0t)
            pltpu.sync_copy(ei.at[1, pl.ds(off, _TAIL)], idx1t)
            pltpu.async_copy(y.at[idx0t], valst, gsem).wait()
            pltpu.async_copy(valst, sum_r.at[idx1t], ssem, add=True).wait()
            pltpu.async_copy(onest, cnt_r.at[idx1t], csem, add=True).wait()

    plsc.subcore_barrier()

    def writeout(acc, out, stripe, last):
        @pl.when(s < _NSUB - 1)
        def _():
            o = s * stripe
            pltpu.sync_copy(acc.at[pl.ds(o, stripe)], wbuf.at[pl.ds(0, stripe)])
            pltpu.sync_copy(wbuf.at[pl.ds(0, stripe)], out.at[pl.ds(o, stripe)])

        @pl.when(s == _NSUB - 1)
        def _():
            o = (_NSUB - 1) * stripe
            pltpu.sync_copy(acc.at[pl.ds(o, last)], wbuf.at[pl.ds(0, last)])
            pltpu.sync_copy(wbuf.at[pl.ds(0, last)], out.at[pl.ds(o, last)])

    for slot in range(7):
        @pl.when(c == 0)
        def _():
            writeout(sums[slot], o_sum_ind[slot], 6256, 6160)
            writeout(cnts[slot], o_cnt_ind[slot], 6256, 6160)

        @pl.when(c == 1)
        def _():
            writeout(sums[slot], o_sum_org[slot], 3128, 3080)
            writeout(cnts[slot], o_cnt_org[slot], 3128, 3080)


def _segment_call(ys, eis, zeros_hbm, ones_hbm):
    mesh = plsc.VectorSubcoreMesh(core_axis_name="c", subcore_axis_name="s",
                                  num_cores=2, num_subcores=_NSUB)
    f = pl.kernel(
        _segment_body,
        out_type=(
            [jax.ShapeDtypeStruct((100000,), jnp.float32)] * 14
            + [jax.ShapeDtypeStruct((50000,), jnp.float32)] * 14
        ),
        mesh=mesh,
        scratch_types=(
            [pltpu.VMEM_SHARED((_NP,), jnp.float32) for _ in range(14)]
            + [pltpu.VMEM((_CH,), jnp.int32),
               pltpu.VMEM((_CH,), jnp.int32),
               pltpu.VMEM((_CH,), jnp.float32),
               pltpu.VMEM((_CH,), jnp.float32),
               pltpu.VMEM((_TAIL,), jnp.int32),
               pltpu.VMEM((_TAIL,), jnp.int32),
               pltpu.VMEM((_TAIL,), jnp.float32),
               pltpu.VMEM((_TAIL,), jnp.float32),
               pltpu.VMEM((_ZCH,), jnp.float32),
               pltpu.VMEM((6256,), jnp.float32)]
            + [pltpu.SemaphoreType.DMA] * 3
        ),
    )
    return f(*ys, *eis, zeros_hbm, ones_hbm)


def kernel(x_ind, x_org, x_ext, ei_ind_txn_ind, ei_org_txn_ind,
           ei_ext_txn_ind, ei_ind_txn_org, ei_org_txn_org, ei_ext_txn_org,
           ei_ind_role_org, ei_ind_rev_txn_ind, ei_org_rev_txn_ind,
           ei_ext_rev_txn_ind, ei_ind_rev_txn_org, ei_org_rev_txn_org,
           ei_ext_rev_txn_org, ei_org_rev_role_ind, edge_attr_dummy,
           Wl, bl, Wr):
    eis = [ei_ind_txn_ind, ei_org_txn_ind, ei_ext_txn_ind, ei_ind_txn_org,
           ei_org_txn_org, ei_ext_txn_org, ei_ind_role_org,
           ei_ind_rev_txn_ind, ei_org_rev_txn_ind, ei_ext_rev_txn_ind,
           ei_ind_rev_txn_org, ei_org_rev_txn_org, ei_ext_rev_txn_org,
           ei_org_rev_role_ind]
    x = {"ind": x_ind, "org": x_org, "ext": x_ext}

    ats = {}
    for t in ("ind", "org", "ext"):
        rows = [jnp.zeros((128,), jnp.float32)] * 8
        for r in range(_NREL):
            if _SRC[r] == t:
                rows[_SRC_COL[r]] = Wl[r, :, 0]
        if t != "ext":
            rows[5] = sum(Wr[r, :, 0] for r in range(_NREL) if _DST[r] == t)
        ats[t] = jnp.stack(rows)

    yt = {t: _project(x[t], ats[t]) for t in ("ind", "org", "ext")}
    ys = [yt[_SRC[r]][_SRC_COL[r]] for r in range(_NREL)]

    zeros_hbm = jnp.zeros((_ZCH,), jnp.float32)
    ones_hbm = jnp.ones((_CH,), jnp.float32)
    outs = _segment_call(ys, eis, zeros_hbm, ones_hbm)
    s_ind, c_ind = outs[0:7], outs[7:14]
    s_org, c_org = outs[14:21], outs[21:28]

    bsum = {t: jnp.sum(jnp.stack(
        [bl[r, 0] for r in range(_NREL) if _DST[r] == t])).reshape(1, 1)
        for t in ("ind", "org")}

    out_ind = _combine(s_ind, c_ind, yt["ind"][5], bsum["ind"])
    out_org = _combine(s_org, c_org, yt["org"][5], bsum["org"])
    return out_ind, out_org


# speedup vs baseline: 18.1153x; 1.8495x over previous
import functools

import jax
import jax.numpy as jnp
from jax import lax
from jax.experimental import pallas as pl
from jax.experimental.pallas import tpu as pltpu
from jax.experimental.pallas import tpu_sc as plsc

_SRC = ["ind", "org", "ext", "ind", "org", "ext", "ind",
        "ind", "org", "ext", "ind", "org", "ext", "org"]
_DST = ["ind", "ind", "ind", "org", "org", "org", "org",
        "ind", "ind", "ind", "org", "org", "org", "ind"]
_NREL = 14

_SRC_COL = {}
for _t in ("ind", "org", "ext"):
    for _c, _r in enumerate([i for i in range(_NREL) if _SRC[i] == _t]):
        _SRC_COL[_r] = _c
_DST_SLOT = {}
for _t in ("ind", "org"):
    for _c, _r in enumerate([i for i in range(_NREL) if _DST[i] == _t]):
        _DST_SLOT[_r] = _c
_CORE = {r: (0 if _DST[r] == "ind" else 1) for r in range(_NREL)}

_E = 40000
_CH = 128
_NFULL = _E // _CH
_TAIL = _E - _NFULL * _CH
_NSUB = 16
_KMAX = -(-_NFULL // _NSUB)
_NP = 106496
_ZCH = 8192


def _project_kernel(a_ref, x_ref, o_ref):
    o_ref[...] = lax.dot_general(
        a_ref[...], x_ref[...], (((1,), (1,)), ((), ())),
        preferred_element_type=jnp.float32)


def _project(x, at, bn=2048):
    n = x.shape[0]
    grid = -(-n // bn)
    return pl.pallas_call(
        _project_kernel,
        grid=(grid,),
        in_specs=[
            pl.BlockSpec((8, 128), lambda i: (0, 0)),
            pl.BlockSpec((bn, 128), lambda i: (i, 0)),
        ],
        out_specs=pl.BlockSpec((8, bn), lambda i: (0, i)),
        out_shape=jax.ShapeDtypeStruct((8, n), jnp.float32),
    )(at, x)


def _combine_kernel(*refs):
    sums = refs[0:7]
    cnts = refs[7:14]
    y_ref, b_ref, o_ref = refs[14], refs[15], refs[16]
    tot = y_ref[...] + b_ref[0, 0]
    for j in range(7):
        tot = tot + sums[j][...] / jnp.maximum(cnts[j][...], 1.0)
    o_ref[...] = jax.nn.sigmoid(tot)


def _combine(sums, cnts, z, bsum, bn=1024):
    n = z.shape[0]
    grid = -(-n // bn)
    vec = pl.BlockSpec((bn,), lambda i: (i,))
    return pl.pallas_call(
        _combine_kernel,
        grid=(grid,),
        in_specs=[vec] * 15 + [pl.BlockSpec(memory_space=pltpu.SMEM)],
        out_specs=vec,
        out_shape=jax.ShapeDtypeStruct((n,), jnp.float32),
    )(*sums, *cnts, z, bsum)


def _segment_body(*refs):
    ys = refs[0:_NREL]
    eis = refs[_NREL:2 * _NREL]
    zeros_hbm = refs[2 * _NREL]
    ones_hbm = refs[2 * _NREL + 1]
    o_sum_ind = refs[30:37]
    o_cnt_ind = refs[37:44]
    o_sum_org = refs[44:51]
    o_cnt_org = refs[51:58]
    sc = refs[58:]
    sums = sc[0:7]
    cnts = sc[7:14]
    idx0_all, idx1_all, vals_all, ones, idx0t, idx1t, valst, onest = sc[14:22]
    zbuf, wbuf = sc[22:24]
    esem, gsem, ssem, csem = sc[24:28]

    c = lax.axis_index("c")
    s = lax.axis_index("s")

    pltpu.sync_copy(ones_hbm, ones)
    pltpu.sync_copy(ones_hbm.at[pl.ds(0, _TAIL)], onest)
    pltpu.sync_copy(zeros_hbm, zbuf)
    for a, acc in enumerate(sums + cnts):
        @pl.loop(0, _NP // _ZCH)
        def _(i):
            @pl.when(((a * (_NP // _ZCH) + i) % _NSUB) == s)
            def _():
                pltpu.sync_copy(zbuf, acc.at[pl.ds(i * _ZCH, _ZCH)])
    plsc.subcore_barrier()

    def for_chunks(fn):
        @pl.loop(0, _KMAX)
        def _(k):
            j = k * _NSUB + s

            @pl.when(j < _NFULL)
            def _():
                fn(k, j)

    for r in range(_NREL):
        y = ys[r]
        ei = eis[r]
        sum_r = sums[_DST_SLOT[r]]
        cnt_r = cnts[_DST_SLOT[r]]
        on_core = c == _CORE[r]
        toff = _NFULL * _CH

        @pl.when(on_core)
        def _():
            def fire_edges(k, j):
                off = j * _CH
                pltpu.async_copy(ei.at[0, pl.ds(off, _CH)], idx0_all.at[k],
                                 esem)
                pltpu.async_copy(ei.at[1, pl.ds(off, _CH)], idx1_all.at[k],
                                 esem)
            for_chunks(fire_edges)

            @pl.when(s == r)
            def _():
                pltpu.async_copy(ei.at[0, pl.ds(toff, _TAIL)], idx0t, esem)
                pltpu.async_copy(ei.at[1, pl.ds(toff, _TAIL)], idx1t, esem)

            def wait_edges(k, j):
                off = j * _CH
                pltpu.make_async_copy(ei.at[0, pl.ds(off, _CH)],
                                      idx0_all.at[k], esem).wait()
                pltpu.make_async_copy(ei.at[1, pl.ds(off, _CH)],
                                      idx1_all.at[k], esem).wait()
            for_chunks(wait_edges)

            @pl.when(s == r)
            def _():
                pltpu.make_async_copy(ei.at[0, pl.ds(toff, _TAIL)], idx0t,
                                      esem).wait()
                pltpu.make_async_copy(ei.at[1, pl.ds(toff, _TAIL)], idx1t,
                                      esem).wait()

            for_chunks(lambda k, j: pltpu.async_copy(
                y.at[idx0_all.at[k]], vals_all.at[k], gsem))

            @pl.when(s == r)
            def _():
                pltpu.async_copy(y.at[idx0t], valst, gsem)

            for_chunks(lambda k, j: pltpu.make_async_copy(
                y.at[idx0_all.at[k]], vals_all.at[k], gsem).wait())

            @pl.when(s == r)
            def _():
                pltpu.make_async_copy(y.at[idx0t], valst, gsem).wait()

            def fire_scatter(k, j):
                pltpu.async_copy(vals_all.at[k], sum_r.at[idx1_all.at[k]],
                                 ssem, add=True)
                pltpu.async_copy(ones, cnt_r.at[idx1_all.at[k]], csem,
                                 add=True)
            for_chunks(fire_scatter)

            @pl.when(s == r)
            def _():
                pltpu.async_copy(valst, sum_r.at[idx1t], ssem, add=True)
                pltpu.async_copy(onest, cnt_r.at[idx1t], csem, add=True)

            def wait_scatter(k, j):
                pltpu.make_async_copy(vals_all.at[k],
                                      sum_r.at[idx1_all.at[k]], ssem).wait()
                pltpu.make_async_copy(ones, cnt_r.at[idx1_all.at[k]],
                                      csem).wait()
            for_chunks(wait_scatter)

            @pl.when(s == r)
            def _():
                pltpu.make_async_copy(valst, sum_r.at[idx1t], ssem).wait()
                pltpu.make_async_copy(onest, cnt_r.at[idx1t], csem).wait()

    plsc.subcore_barrier()

    def writeout(acc, out, stripe, last):
        @pl.when(s < _NSUB - 1)
        def _():
            o = s * stripe
            pltpu.sync_copy(acc.at[pl.ds(o, stripe)], wbuf.at[pl.ds(0, stripe)])
            pltpu.sync_copy(wbuf.at[pl.ds(0, stripe)], out.at[pl.ds(o, stripe)])

        @pl.when(s == _NSUB - 1)
        def _():
            o = (_NSUB - 1) * stripe
            pltpu.sync_copy(acc.at[pl.ds(o, last)], wbuf.at[pl.ds(0, last)])
            pltpu.sync_copy(wbuf.at[pl.ds(0, last)], out.at[pl.ds(o, last)])

    for slot in range(7):
        @pl.when(c == 0)
        def _():
            writeout(sums[slot], o_sum_ind[slot], 6256, 6160)
            writeout(cnts[slot], o_cnt_ind[slot], 6256, 6160)

        @pl.when(c == 1)
        def _():
            writeout(sums[slot], o_sum_org[slot], 3128, 3080)
            writeout(cnts[slot], o_cnt_org[slot], 3128, 3080)


def _segment_call(ys, eis, zeros_hbm, ones_hbm):
    mesh = plsc.VectorSubcoreMesh(core_axis_name="c", subcore_axis_name="s",
                                  num_cores=2, num_subcores=_NSUB)
    f = pl.kernel(
        _segment_body,
        out_type=(
            [jax.ShapeDtypeStruct((100000,), jnp.float32)] * 14
            + [jax.ShapeDtypeStruct((50000,), jnp.float32)] * 14
        ),
        mesh=mesh,
        scratch_types=(
            [pltpu.VMEM_SHARED((_NP,), jnp.float32) for _ in range(14)]
            + [pltpu.VMEM((_KMAX, _CH), jnp.int32),
               pltpu.VMEM((_KMAX, _CH), jnp.int32),
               pltpu.VMEM((_KMAX, _CH), jnp.float32),
               pltpu.VMEM((_CH,), jnp.float32),
               pltpu.VMEM((_TAIL,), jnp.int32),
               pltpu.VMEM((_TAIL,), jnp.int32),
               pltpu.VMEM((_TAIL,), jnp.float32),
               pltpu.VMEM((_TAIL,), jnp.float32),
               pltpu.VMEM((_ZCH,), jnp.float32),
               pltpu.VMEM((6256,), jnp.float32)]
            + [pltpu.SemaphoreType.DMA] * 4
        ),
    )
    return f(*ys, *eis, zeros_hbm, ones_hbm)


def kernel(x_ind, x_org, x_ext, ei_ind_txn_ind, ei_org_txn_ind,
           ei_ext_txn_ind, ei_ind_txn_org, ei_org_txn_org, ei_ext_txn_org,
           ei_ind_role_org, ei_ind_rev_txn_ind, ei_org_rev_txn_ind,
           ei_ext_rev_txn_ind, ei_ind_rev_txn_org, ei_org_rev_txn_org,
           ei_ext_rev_txn_org, ei_org_rev_role_ind, edge_attr_dummy,
           Wl, bl, Wr):
    eis = [ei_ind_txn_ind, ei_org_txn_ind, ei_ext_txn_ind, ei_ind_txn_org,
           ei_org_txn_org, ei_ext_txn_org, ei_ind_role_org,
           ei_ind_rev_txn_ind, ei_org_rev_txn_ind, ei_ext_rev_txn_ind,
           ei_ind_rev_txn_org, ei_org_rev_txn_org, ei_ext_rev_txn_org,
           ei_org_rev_role_ind]
    x = {"ind": x_ind, "org": x_org, "ext": x_ext}

    ats = {}
    for t in ("ind", "org", "ext"):
        rows = [jnp.zeros((128,), jnp.float32)] * 8
        for r in range(_NREL):
            if _SRC[r] == t:
                rows[_SRC_COL[r]] = Wl[r, :, 0]
        if t != "ext":
            rows[5] = sum(Wr[r, :, 0] for r in range(_NREL) if _DST[r] == t)
        ats[t] = jnp.stack(rows)

    yt = {t: _project(x[t], ats[t]) for t in ("ind", "org", "ext")}
    ys = [yt[_SRC[r]][_SRC_COL[r]] for r in range(_NREL)]

    zeros_hbm = jnp.zeros((_ZCH,), jnp.float32)
    ones_hbm = jnp.ones((_CH,), jnp.float32)
    outs = _segment_call(ys, eis, zeros_hbm, ones_hbm)
    s_ind, c_ind = outs[0:7], outs[7:14]
    s_org, c_org = outs[14:21], outs[21:28]

    bsum = {t: jnp.sum(jnp.stack(
        [bl[r, 0] for r in range(_NREL) if _DST[r] == t])).reshape(1, 1)
        for t in ("ind", "org")}

    out_ind = _combine(s_ind, c_ind, yt["ind"][5], bsum["ind"])
    out_org = _combine(s_org, c_org, yt["org"][5], bsum["org"])
    return out_ind, out_org


# speedup vs baseline: 27.8427x; 1.5370x over previous
import functools

import jax
import jax.numpy as jnp
from jax import lax
from jax.experimental import pallas as pl
from jax.experimental.pallas import tpu as pltpu
from jax.experimental.pallas import tpu_sc as plsc

_SRC = ["ind", "org", "ext", "ind", "org", "ext", "ind",
        "ind", "org", "ext", "ind", "org", "ext", "org"]
_DST = ["ind", "ind", "ind", "org", "org", "org", "org",
        "ind", "ind", "ind", "org", "org", "org", "ind"]
_NREL = 14

_SRC_COL = {}
for _t in ("ind", "org", "ext"):
    for _c, _r in enumerate([i for i in range(_NREL) if _SRC[i] == _t]):
        _SRC_COL[_r] = _c
_DST_SLOT = {}
for _t in ("ind", "org"):
    for _c, _r in enumerate([i for i in range(_NREL) if _DST[i] == _t]):
        _DST_SLOT[_r] = _c
_CORE = {r: (0 if _DST[r] == "ind" else 1) for r in range(_NREL)}

_E = 40000
_CH = 128
_NFULL = _E // _CH
_TAIL = _E - _NFULL * _CH
_NSUB = 16
_KMAX = -(-_NFULL // _NSUB)
_NP = 106496
_ZCH = 8192


def _project_kernel(a_ref, x_ref, *o_refs):
    res = lax.dot_general(
        a_ref[...], x_ref[...], (((1,), (1,)), ((), ())),
        preferred_element_type=jnp.float32)
    for j, o_ref in enumerate(o_refs):
        o_ref[...] = res[j, :]


def _project(x, at, ncols, bn=2048):
    n = x.shape[0]
    grid = -(-n // bn)
    vec = pl.BlockSpec((bn,), lambda i: (i,))
    return pl.pallas_call(
        _project_kernel,
        grid=(grid,),
        in_specs=[
            pl.BlockSpec((8, 128), lambda i: (0, 0)),
            pl.BlockSpec((bn, 128), lambda i: (i, 0)),
        ],
        out_specs=[vec] * ncols,
        out_shape=[jax.ShapeDtypeStruct((n,), jnp.float32)] * ncols,
    )(at, x)


def _combine_kernel(*refs):
    sums = refs[0:7]
    cnts = refs[7:14]
    y_ref, b_ref, o_ref = refs[14], refs[15], refs[16]
    tot = y_ref[...] + b_ref[0, 0]
    for j in range(7):
        tot = tot + sums[j][...] / jnp.maximum(cnts[j][...], 1.0)
    o_ref[...] = jax.nn.sigmoid(tot)


def _combine(sums, cnts, z, bsum, bn=8192):
    n = z.shape[0]
    grid = -(-n // bn)
    vec = pl.BlockSpec((bn,), lambda i: (i,))
    return pl.pallas_call(
        _combine_kernel,
        grid=(grid,),
        in_specs=[vec] * 15 + [pl.BlockSpec(memory_space=pltpu.SMEM)],
        out_specs=vec,
        out_shape=jax.ShapeDtypeStruct((n,), jnp.float32),
    )(*sums, *cnts, z, bsum)


def _segment_body(*refs):
    ys = refs[0:_NREL]
    eis = refs[_NREL:2 * _NREL]
    zeros_hbm = refs[28]
    ones_hbm = refs[29]
    o_sum_ind = refs[30:37]
    o_cnt_ind = refs[37:44]
    o_sum_org = refs[44:51]
    o_cnt_org = refs[51:58]
    sc = refs[58:]
    sums = sc[0:7]
    cnts = sc[7:14]
    idx0_all, idx1_all, vals_all, ones, idx0t, idx1t, valst, onest = sc[14:22]
    zbuf, wbuf = sc[22:24]
    esem, gsem, ssem, csem = sc[24:28]

    c = lax.axis_index("c")
    s = lax.axis_index("s")

    pltpu.sync_copy(ones_hbm, ones)
    pltpu.sync_copy(ones_hbm.at[pl.ds(0, _TAIL)], onest)
    pltpu.sync_copy(zeros_hbm, zbuf)
    for a, acc in enumerate(sums + cnts):
        @pl.loop(0, _NP // _ZCH)
        def _(i):
            @pl.when(((a * (_NP // _ZCH) + i) % _NSUB) == s)
            def _():
                pltpu.sync_copy(zbuf, acc.at[pl.ds(i * _ZCH, _ZCH)])
    plsc.subcore_barrier()

    def for_chunks(fn):
        @pl.loop(0, _KMAX)
        def _(k):
            j = k * _NSUB + s

            @pl.when(j < _NFULL)
            def _():
                fn(k, j)

    for r in range(_NREL):
        y = ys[r]
        ei = eis[r]
        sum_r = sums[_DST_SLOT[r]]
        cnt_r = cnts[_DST_SLOT[r]]
        on_core = c == _CORE[r]
        toff = _NFULL * _CH

        @pl.when(on_core)
        def _():
            def fire_edges(k, j):
                off = j * _CH
                pltpu.async_copy(ei.at[0, pl.ds(off, _CH)], idx0_all.at[k],
                                 esem)
                pltpu.async_copy(ei.at[1, pl.ds(off, _CH)], idx1_all.at[k],
                                 esem)
            for_chunks(fire_edges)

            @pl.when(s == r)
            def _():
                pltpu.async_copy(ei.at[0, pl.ds(toff, _TAIL)], idx0t, esem)
                pltpu.async_copy(ei.at[1, pl.ds(toff, _TAIL)], idx1t, esem)

            def wait_edges(k, j):
                off = j * _CH
                pltpu.make_async_copy(ei.at[0, pl.ds(off, _CH)],
                                      idx0_all.at[k], esem).wait()
                pltpu.make_async_copy(ei.at[1, pl.ds(off, _CH)],
                                      idx1_all.at[k], esem).wait()
            for_chunks(wait_edges)

            @pl.when(s == r)
            def _():
                pltpu.make_async_copy(ei.at[0, pl.ds(toff, _TAIL)], idx0t,
                                      esem).wait()
                pltpu.make_async_copy(ei.at[1, pl.ds(toff, _TAIL)], idx1t,
                                      esem).wait()

            for_chunks(lambda k, j: pltpu.async_copy(
                y.at[idx0_all.at[k]], vals_all.at[k], gsem))

            @pl.when(s == r)
            def _():
                pltpu.async_copy(y.at[idx0t], valst, gsem)

            for_chunks(lambda k, j: pltpu.make_async_copy(
                y.at[idx0_all.at[k]], vals_all.at[k], gsem).wait())

            @pl.when(s == r)
            def _():
                pltpu.make_async_copy(y.at[idx0t], valst, gsem).wait()

            def fire_scatter(k, j):
                pltpu.async_copy(vals_all.at[k], sum_r.at[idx1_all.at[k]],
                                 ssem, add=True)
                pltpu.async_copy(ones, cnt_r.at[idx1_all.at[k]], csem,
                                 add=True)
            for_chunks(fire_scatter)

            @pl.when(s == r)
            def _():
                pltpu.async_copy(valst, sum_r.at[idx1t], ssem, add=True)
                pltpu.async_copy(onest, cnt_r.at[idx1t], csem, add=True)

            def wait_scatter(k, j):
                pltpu.make_async_copy(vals_all.at[k],
                                      sum_r.at[idx1_all.at[k]], ssem).wait()
                pltpu.make_async_copy(ones, cnt_r.at[idx1_all.at[k]],
                                      csem).wait()
            for_chunks(wait_scatter)

            @pl.when(s == r)
            def _():
                pltpu.make_async_copy(valst, sum_r.at[idx1t], ssem).wait()
                pltpu.make_async_copy(onest, cnt_r.at[idx1t], csem).wait()

    plsc.subcore_barrier()

    def writeout(acc, out, stripe, last):
        @pl.when(s < _NSUB - 1)
        def _():
            o = s * stripe
            pltpu.sync_copy(acc.at[pl.ds(o, stripe)], wbuf.at[pl.ds(0, stripe)])
            pltpu.sync_copy(wbuf.at[pl.ds(0, stripe)], out.at[pl.ds(o, stripe)])

        @pl.when(s == _NSUB - 1)
        def _():
            o = (_NSUB - 1) * stripe
            pltpu.sync_copy(acc.at[pl.ds(o, last)], wbuf.at[pl.ds(0, last)])
            pltpu.sync_copy(wbuf.at[pl.ds(0, last)], out.at[pl.ds(o, last)])

    for slot in range(7):
        @pl.when(c == 0)
        def _():
            writeout(sums[slot], o_sum_ind[slot], 6256, 6160)
            writeout(cnts[slot], o_cnt_ind[slot], 6256, 6160)

        @pl.when(c == 1)
        def _():
            writeout(sums[slot], o_sum_org[slot], 3128, 3080)
            writeout(cnts[slot], o_cnt_org[slot], 3128, 3080)


def _segment_call(ys, eis, zeros_hbm, ones_hbm):
    mesh = plsc.VectorSubcoreMesh(core_axis_name="c", subcore_axis_name="s",
                                  num_cores=2, num_subcores=_NSUB)
    f = pl.kernel(
        _segment_body,
        out_type=(
            [jax.ShapeDtypeStruct((100000,), jnp.float32)] * 14
            + [jax.ShapeDtypeStruct((50000,), jnp.float32)] * 14
        ),
        mesh=mesh,
        scratch_types=(
            [pltpu.VMEM_SHARED((_NP,), jnp.float32) for _ in range(14)]
            + [pltpu.VMEM((_KMAX, _CH), jnp.int32),
               pltpu.VMEM((_KMAX, _CH), jnp.int32),
               pltpu.VMEM((_KMAX, _CH), jnp.float32),
               pltpu.VMEM((_CH,), jnp.float32),
               pltpu.VMEM((_TAIL,), jnp.int32),
               pltpu.VMEM((_TAIL,), jnp.int32),
               pltpu.VMEM((_TAIL,), jnp.float32),
               pltpu.VMEM((_TAIL,), jnp.float32),
               pltpu.VMEM((_ZCH,), jnp.float32),
               pltpu.VMEM((6256,), jnp.float32)]
            + [pltpu.SemaphoreType.DMA] * 4
        ),
    )
    return f(*ys, *eis, zeros_hbm, ones_hbm)


def kernel(x_ind, x_org, x_ext, ei_ind_txn_ind, ei_org_txn_ind,
           ei_ext_txn_ind, ei_ind_txn_org, ei_org_txn_org, ei_ext_txn_org,
           ei_ind_role_org, ei_ind_rev_txn_ind, ei_org_rev_txn_ind,
           ei_ext_rev_txn_ind, ei_ind_rev_txn_org, ei_org_rev_txn_org,
           ei_ext_rev_txn_org, ei_org_rev_role_ind, edge_attr_dummy,
           Wl, bl, Wr):
    eis = [ei_ind_txn_ind, ei_org_txn_ind, ei_ext_txn_ind, ei_ind_txn_org,
           ei_org_txn_org, ei_ext_txn_org, ei_ind_role_org,
           ei_ind_rev_txn_ind, ei_org_rev_txn_ind, ei_ext_rev_txn_ind,
           ei_ind_rev_txn_org, ei_org_rev_txn_org, ei_ext_rev_txn_org,
           ei_org_rev_role_ind]
    x = {"ind": x_ind, "org": x_org, "ext": x_ext}

    ats = {}
    for t in ("ind", "org", "ext"):
        rows = [jnp.zeros((128,), jnp.float32)] * 8
        for r in range(_NREL):
            if _SRC[r] == t:
                rows[_SRC_COL[r]] = Wl[r, :, 0]
        if t != "ext":
            rows[5] = sum(Wr[r, :, 0] for r in range(_NREL) if _DST[r] == t)
        ats[t] = jnp.stack(rows)

    yt = {t: _project(x[t], ats[t], 4 if t == "ext" else 6)
          for t in ("ind", "org", "ext")}
    ys = [yt[_SRC[r]][_SRC_COL[r]] for r in range(_NREL)]

    zeros_hbm = jnp.zeros((_ZCH,), jnp.float32)
    ones_hbm = jnp.ones((_CH,), jnp.float32)
    outs = _segment_call(ys, eis, zeros_hbm, ones_hbm)
    s_ind, c_ind = outs[0:7], outs[7:14]
    s_org, c_org = outs[14:21], outs[21:28]

    bsum = {t: jnp.sum(jnp.stack(
        [bl[r, 0] for r in range(_NREL) if _DST[r] == t])).reshape(1, 1)
        for t in ("ind", "org")}

    out_ind = _combine(s_ind, c_ind, yt["ind"][5], bsum["ind"])
    out_org = _combine(s_org, c_org, yt["org"][5], bsum["org"])
    return out_ind, out_org


# speedup vs baseline: 36.9801x; 1.3282x over previous
import functools

import jax
import jax.numpy as jnp
from jax import lax
from jax.experimental import pallas as pl
from jax.experimental.pallas import tpu as pltpu
from jax.experimental.pallas import tpu_sc as plsc

_SRC = ["ind", "org", "ext", "ind", "org", "ext", "ind",
        "ind", "org", "ext", "ind", "org", "ext", "org"]
_DST = ["ind", "ind", "ind", "org", "org", "org", "org",
        "ind", "ind", "ind", "org", "org", "org", "ind"]
_NREL = 14

_SRC_COL = {}
for _t in ("ind", "org", "ext"):
    for _c, _r in enumerate([i for i in range(_NREL) if _SRC[i] == _t]):
        _SRC_COL[_r] = _c
_DST_SLOT = {}
for _t in ("ind", "org"):
    for _c, _r in enumerate([i for i in range(_NREL) if _DST[i] == _t]):
        _DST_SLOT[_r] = _c
_CORE = {r: (0 if _DST[r] == "ind" else 1) for r in range(_NREL)}

_E = 40000
_CH = 128
_NFULL = _E // _CH
_TAIL = _E - _NFULL * _CH
_NSUB = 16
_KMAX = -(-_NFULL // _NSUB)
_NP = 102400
_ZCH = 2048


def _project_kernel(a_ref, x_ref, *o_refs):
    res = lax.dot_general(
        a_ref[...], x_ref[...], (((1,), (1,)), ((), ())),
        preferred_element_type=jnp.float32)
    for j, o_ref in enumerate(o_refs):
        o_ref[...] = res[j, :]


def _project(x, at, ncols, bn=8192):
    n = x.shape[0]
    grid = -(-n // bn)
    vec = pl.BlockSpec((bn,), lambda i: (i,))
    return pl.pallas_call(
        _project_kernel,
        grid=(grid,),
        in_specs=[
            pl.BlockSpec((8, 128), lambda i: (0, 0)),
            pl.BlockSpec((bn, 128), lambda i: (i, 0)),
        ],
        out_specs=[vec] * ncols,
        out_shape=[jax.ShapeDtypeStruct((n,), jnp.float32)] * ncols,
    )(at, x)


def _combine_kernel(*refs):
    sums = refs[0:7]
    cnts = refs[7:14]
    y_ref, b_ref, o_ref = refs[14], refs[15], refs[16]
    tot = y_ref[...] + b_ref[0, 0]
    for j in range(7):
        tot = tot + sums[j][...] / jnp.maximum(cnts[j][...], 1.0)
    o_ref[...] = jax.nn.sigmoid(tot)


def _combine(sums, cnts, z, bsum, bn=8192):
    n = z.shape[0]
    grid = -(-n // bn)
    vec = pl.BlockSpec((bn,), lambda i: (i,))
    return pl.pallas_call(
        _combine_kernel,
        grid=(grid,),
        in_specs=[vec] * 15 + [pl.BlockSpec(memory_space=pltpu.SMEM)],
        out_specs=vec,
        out_shape=jax.ShapeDtypeStruct((n,), jnp.float32),
    )(*sums, *cnts, z, bsum)


def _segment_body(*refs):
    ys = refs[0:_NREL]
    eis = refs[_NREL:2 * _NREL]
    zeros_hbm = refs[28]
    ones_hbm = refs[29]
    o_sum_ind = refs[30:37]
    o_cnt_ind = refs[37:44]
    o_sum_org = refs[44:51]
    o_cnt_org = refs[51:58]
    sc = refs[58:]
    sums = sc[0:7]
    cnts = sc[7:14]
    idx0_all, idx1_all, vals_all, ones, idx0t, idx1t, valst, onest = sc[14:22]
    wbufa, wbufb = sc[22:24]
    zbuf = wbufa.at[pl.ds(0, _ZCH)]
    es = sc[24:27]
    gs = sc[27:30]
    ss = sc[30:33]
    cs = sc[33:36]
    zsem = sc[36]

    c = lax.axis_index("c")
    s = lax.axis_index("s")

    core_rels = ([r for r in range(_NREL) if _CORE[r] == 0],
                 [r for r in range(_NREL) if _CORE[r] == 1])
    toff = _NFULL * _CH

    def for_chunks(fn):
        @pl.loop(0, _KMAX)
        def _(k):
            j = k * _NSUB + s

            @pl.when(j < _NFULL)
            def _():
                fn(k, j)

    def fire_edges(i, p):
        for r in (core_rels[0][i], core_rels[1][i]):
            ei = eis[r]

            @pl.when(c == _CORE[r])
            def _():
                def f(k, j):
                    off = j * _CH
                    pltpu.async_copy(ei.at[0, pl.ds(off, _CH)],
                                     idx0_all.at[p, k], es[p])
                    pltpu.async_copy(ei.at[1, pl.ds(off, _CH)],
                                     idx1_all.at[p, k], es[p])
                for_chunks(f)

                @pl.when(s == r)
                def _():
                    pltpu.async_copy(ei.at[0, pl.ds(toff, _TAIL)], idx0t, es[p])
                    pltpu.async_copy(ei.at[1, pl.ds(toff, _TAIL)], idx1t, es[p])

    def drain_edges(i, p):
        for r in (core_rels[0][i], core_rels[1][i]):
            ei = eis[r]

            @pl.when(c == _CORE[r])
            def _():
                def f(k, j):
                    off = j * _CH
                    pltpu.make_async_copy(ei.at[0, pl.ds(off, _CH)],
                                          idx0_all.at[p, k], es[p]).wait()
                    pltpu.make_async_copy(ei.at[1, pl.ds(off, _CH)],
                                          idx1_all.at[p, k], es[p]).wait()
                for_chunks(f)

                @pl.when(s == r)
                def _():
                    pltpu.make_async_copy(ei.at[0, pl.ds(toff, _TAIL)],
                                          idx0t, es[p]).wait()
                    pltpu.make_async_copy(ei.at[1, pl.ds(toff, _TAIL)],
                                          idx1t, es[p]).wait()

    def fire_gather(i, p):
        for r in (core_rels[0][i], core_rels[1][i]):
            y = ys[r]

            @pl.when(c == _CORE[r])
            def _():
                for_chunks(lambda k, j: pltpu.async_copy(
                    y.at[idx0_all.at[p, k]], vals_all.at[p, k], gs[p]))

                @pl.when(s == r)
                def _():
                    pltpu.async_copy(y.at[idx0t], valst, gs[p])

    def drain_gather(i, p):
        for r in (core_rels[0][i], core_rels[1][i]):
            y = ys[r]

            @pl.when(c == _CORE[r])
            def _():
                for_chunks(lambda k, j: pltpu.make_async_copy(
                    y.at[idx0_all.at[p, k]], vals_all.at[p, k], gs[p]).wait())

                @pl.when(s == r)
                def _():
                    pltpu.make_async_copy(y.at[idx0t], valst, gs[p]).wait()

    def fire_scatter(i, p):
        for r in (core_rels[0][i], core_rels[1][i]):
            sum_r = sums[_DST_SLOT[r]]
            cnt_r = cnts[_DST_SLOT[r]]

            @pl.when(c == _CORE[r])
            def _():
                def f(k, j):
                    pltpu.async_copy(vals_all.at[p, k],
                                     sum_r.at[idx1_all.at[p, k]], ss[p],
                                     add=True)
                    pltpu.async_copy(ones, cnt_r.at[idx1_all.at[p, k]], cs[p],
                                     add=True)
                for_chunks(f)

                @pl.when(s == r)
                def _():
                    pltpu.async_copy(valst, sum_r.at[idx1t], ss[p], add=True)
                    pltpu.async_copy(onest, cnt_r.at[idx1t], cs[p], add=True)

    def drain_scatter(i, p):
        for r in (core_rels[0][i], core_rels[1][i]):
            sum_r = sums[_DST_SLOT[r]]
            cnt_r = cnts[_DST_SLOT[r]]

            @pl.when(c == _CORE[r])
            def _():
                def f(k, j):
                    pltpu.make_async_copy(vals_all.at[p, k],
                                          sum_r.at[idx1_all.at[p, k]],
                                          ss[p]).wait()
                    pltpu.make_async_copy(ones, cnt_r.at[idx1_all.at[p, k]],
                                          cs[p]).wait()
                for_chunks(f)

                @pl.when(s == r)
                def _():
                    pltpu.make_async_copy(valst, sum_r.at[idx1t], ss[p]).wait()
                    pltpu.make_async_copy(onest, cnt_r.at[idx1t], cs[p]).wait()

    fire_edges(0, 0)
    pltpu.sync_copy(ones_hbm, ones)
    pltpu.sync_copy(ones_hbm.at[pl.ds(0, _TAIL)], onest)
    pltpu.sync_copy(zeros_hbm, zbuf)
    nz = _NP // _ZCH
    for a, acc in enumerate(sums + cnts):
        @pl.loop(0, nz)
        def _(i):
            @pl.when(((a * nz + i) % _NSUB) == s)
            def _():
                pltpu.async_copy(zbuf, acc.at[pl.ds(i * _ZCH, _ZCH)], zsem)
    for a, acc in enumerate(sums + cnts):
        @pl.loop(0, nz)
        def _(i):
            @pl.when(((a * nz + i) % _NSUB) == s)
            def _():
                pltpu.make_async_copy(zbuf, acc.at[pl.ds(i * _ZCH, _ZCH)],
                                      zsem).wait()
    plsc.subcore_barrier()

    for i in range(7):
        p = i % 3
        if i + 1 < 7:
            fire_edges(i + 1, (i + 1) % 3)
        drain_edges(i, p)
        fire_gather(i, p)
        drain_gather(i, p)
        fire_scatter(i, p)
        if i >= 1:
            drain_scatter(i - 1, (i - 1) % 3)
    drain_scatter(6, 6 % 3)

    plsc.subcore_barrier()

    def writeout_core(core, o_sum, o_cnt, stripe, last):
        seq = []
        for slot in range(7):
            seq.append((sums[slot], o_sum[slot]))
            seq.append((cnts[slot], o_cnt[slot]))
        bufs = (wbufa, wbufb)
        osems = (ss[0], ss[1])

        def pieces(t, sz, off):
            acc, out = seq[t]
            return (acc.at[pl.ds(off, sz)], bufs[t % 2].at[pl.ds(0, sz)],
                    out.at[pl.ds(off, sz)], osems[t % 2])

        def both_sizes(t, fn):
            @pl.when((c == core) & (s < _NSUB - 1))
            def _():
                fn(*pieces(t, stripe, s * stripe))

            @pl.when((c == core) & (s == _NSUB - 1))
            def _():
                fn(*pieces(t, last, (_NSUB - 1) * stripe))

        def drain_out(a, b, o, sem):
            pltpu.make_async_copy(b, o, sem).wait()

        def move(a, b, o, sem):
            pltpu.sync_copy(a, b)
            pltpu.async_copy(b, o, sem)

        for t in range(len(seq)):
            if t >= 2:
                both_sizes(t - 2, drain_out)
            both_sizes(t, move)
        both_sizes(len(seq) - 2, drain_out)
        both_sizes(len(seq) - 1, drain_out)

    writeout_core(0, o_sum_ind, o_cnt_ind, 6256, 6160)
    writeout_core(1, o_sum_org, o_cnt_org, 3128, 3080)


def _segment_call(ys, eis, zeros_hbm, ones_hbm):
    mesh = plsc.VectorSubcoreMesh(core_axis_name="c", subcore_axis_name="s",
                                  num_cores=2, num_subcores=_NSUB)
    f = pl.kernel(
        _segment_body,
        out_type=(
            [jax.ShapeDtypeStruct((100000,), jnp.float32)] * 14
            + [jax.ShapeDtypeStruct((50000,), jnp.float32)] * 14
        ),
        mesh=mesh,
        scratch_types=(
            [pltpu.VMEM_SHARED((_NP,), jnp.float32) for _ in range(14)]
            + [pltpu.VMEM((3, _KMAX, _CH), jnp.int32),
               pltpu.VMEM((3, _KMAX, _CH), jnp.int32),
               pltpu.VMEM((3, _KMAX, _CH), jnp.float32),
               pltpu.VMEM((_CH,), jnp.float32),
               pltpu.VMEM((_TAIL,), jnp.int32),
               pltpu.VMEM((_TAIL,), jnp.int32),
               pltpu.VMEM((_TAIL,), jnp.float32),
               pltpu.VMEM((_TAIL,), jnp.float32),
               pltpu.VMEM((6256,), jnp.float32),
               pltpu.VMEM((6256,), jnp.float32)]
            + [pltpu.SemaphoreType.DMA] * 13
        ),
    )
    return f(*ys, *eis, zeros_hbm, ones_hbm)


def kernel(x_ind, x_org, x_ext, ei_ind_txn_ind, ei_org_txn_ind,
           ei_ext_txn_ind, ei_ind_txn_org, ei_org_txn_org, ei_ext_txn_org,
           ei_ind_role_org, ei_ind_rev_txn_ind, ei_org_rev_txn_ind,
           ei_ext_rev_txn_ind, ei_ind_rev_txn_org, ei_org_rev_txn_org,
           ei_ext_rev_txn_org, ei_org_rev_role_ind, edge_attr_dummy,
           Wl, bl, Wr):
    eis = [ei_ind_txn_ind, ei_org_txn_ind, ei_ext_txn_ind, ei_ind_txn_org,
           ei_org_txn_org, ei_ext_txn_org, ei_ind_role_org,
           ei_ind_rev_txn_ind, ei_org_rev_txn_ind, ei_ext_rev_txn_ind,
           ei_ind_rev_txn_org, ei_org_rev_txn_org, ei_ext_rev_txn_org,
           ei_org_rev_role_ind]
    x = {"ind": x_ind, "org": x_org, "ext": x_ext}

    ats = {}
    for t in ("ind", "org", "ext"):
        rows = [jnp.zeros((128,), jnp.float32)] * 8
        for r in range(_NREL):
            if _SRC[r] == t:
                rows[_SRC_COL[r]] = Wl[r, :, 0]
        if t != "ext":
            rows[5] = sum(Wr[r, :, 0] for r in range(_NREL) if _DST[r] == t)
        ats[t] = jnp.stack(rows)

    yt = {t: _project(x[t], ats[t], 4 if t == "ext" else 6)
          for t in ("ind", "org", "ext")}
    ys = [yt[_SRC[r]][_SRC_COL[r]] for r in range(_NREL)]

    zeros_hbm = jnp.zeros((_ZCH,), jnp.float32)
    ones_hbm = jnp.ones((_CH,), jnp.float32)
    outs = _segment_call(ys, eis, zeros_hbm, ones_hbm)
    s_ind, c_ind = outs[0:7], outs[7:14]
    s_org, c_org = outs[14:21], outs[21:28]

    bsum = {t: jnp.sum(jnp.stack(
        [bl[r, 0] for r in range(_NREL) if _DST[r] == t])).reshape(1, 1)
        for t in ("ind", "org")}

    out_ind = _combine(s_ind, c_ind, yt["ind"][5], bsum["ind"])
    out_org = _combine(s_org, c_org, yt["org"][5], bsum["org"])
    return out_ind, out_org
